# baseline (speedup 1.0000x reference)
import functools

import jax
import jax.numpy as jnp
from jax import lax
from jax.experimental import pallas as pl
from jax.experimental.pallas import tpu as pltpu
from jax.experimental.pallas import tpu_sc as plsc

_B, _Q, _C = 64, 2048, 256
_R = _B * _Q

_NC, _NS = 2, 16
_NW = _NC * _NS
_RS = 16384
_RW = _RS // _NW
_CH = 128
_NCHUNK = _RW // _CH
_GROUPS = _CH // 16
_UNROLL = 4


def _sc_body(lg_hbm, bx_hbm, lab_hbm, bout_hbm, cnt_hbm,
             lbuf_a, lbuf_b, bxbuf_a, bxbuf_b, labbuf_a, labbuf_b,
             boutbuf_a, boutbuf_b, cntbuf,
             sem_in0, sem_in1, sem_out0, sem_out1):
    cid = lax.axis_index("c")
    sid = lax.axis_index("s")
    wid = sid * _NC + cid
    base_row = wid * _RW

    iot = lax.iota(jnp.int32, 16)
    riot = lax.shift_right_logical(iot, 2)
    neg_inf = jnp.full((16,), -jnp.inf, jnp.float32)
    zero_f = jnp.zeros((16,), jnp.float32)
    zero_i = jnp.zeros((16,), jnp.int32)
    neg1 = jnp.full((16,), -1, jnp.int32)

    sem_in = (sem_in0, sem_in1)
    sem_out = (sem_out0, sem_out1)
    lbufs = (lbuf_a, lbuf_b)
    bxbufs = (bxbuf_a, bxbuf_b)
    labbufs = (labbuf_a, labbuf_b)
    boutbufs = (boutbuf_a, boutbuf_b)

    def start_in(chunk, b):
        row0 = base_row + chunk * _CH
        pltpu.async_copy(lg_hbm.at[pl.ds(row0 * _C, _CH * _C)],
                         lbufs[b], sem_in[b])
        pltpu.async_copy(bx_hbm.at[pl.ds(row0 * 4, _CH * 4)],
                         bxbufs[b], sem_in[b])

    def wait_in(b):
        pltpu.make_async_copy(lg_hbm.at[pl.ds(0, _CH * _C)],
                              lbufs[b], sem_in[b]).wait()
        pltpu.make_async_copy(bx_hbm.at[pl.ds(0, _CH * 4)],
                              bxbufs[b], sem_in[b]).wait()

    def start_out(chunk, b):
        row0 = base_row + chunk * _CH
        pltpu.async_copy(labbufs[b], lab_hbm.at[pl.ds(row0, _CH)],
                         sem_out[b])
        pltpu.async_copy(boutbufs[b], bout_hbm.at[pl.ds(row0 * 4, _CH * 4)],
                         sem_out[b])

    def wait_out(b):
        pltpu.make_async_copy(labbufs[b], lab_hbm.at[pl.ds(0, _CH)],
                              sem_out[b]).wait()
        pltpu.make_async_copy(boutbufs[b], bout_hbm.at[pl.ds(0, _CH * 4)],
                              sem_out[b]).wait()

    start_in(0, 0)
    start_in(1, 1)

    def pair_body(ci2, acc):
        for b in range(2):
            chunk = ci2 * 2 + b
            lbuf = lbufs[b]
            labbuf = labbufs[b]
            boutbuf = boutbufs[b]
            bxbuf = bxbufs[b]

            wait_in(b)
            @pl.when(ci2 > 0)
            def _():
                wait_out(b)

            bvecs = [(g * 16 + iot) * _C for g in range(_GROUPS)]
            init = tuple((neg_inf, bvecs[g], bvecs[g])
                         for g in range(_GROUPS))

            def j_body(_, carry):
                out = []
                for g in range(_GROUPS):
                    best, bidxf, idxv = carry[g]
                    for _u in range(_UNROLL):
                        v = plsc.load_gather(lbuf, [idxv])
                        upd = v > best
                        best = jnp.where(upd, v, best)
                        bidxf = jnp.where(upd, idxv, bidxf)
                        idxv = idxv + 1
                    out.append((best, bidxf, idxv))
                return tuple(out)

            carry = lax.fori_loop(0, _C // _UNROLL, j_body, init)

            for g in range(_GROUPS):
                best, bidxf, _ = carry[g]
                cls = bidxf - bvecs[g]
                valid = best > zero_f
                labbuf[pl.ds(g * 16, 16)] = jnp.where(valid, cls, neg1)
                acc = acc + plsc.all_reduce_population_count(valid)

                for i in range(4):
                    ridx = (g * 16 + 4 * i) + riot
                    lv = plsc.load_gather(labbuf, [ridx])
                    bx = bxbuf[pl.ds(g * 64 + i * 16, 16)]
                    boutbuf[pl.ds(g * 64 + i * 16, 16)] = jnp.where(
                        lv >= zero_i, bx, zero_f)

            start_out(chunk, b)

            @pl.when(chunk + 2 < _NCHUNK)
            def _():
                start_in(chunk + 2, b)
        return acc

    acc = lax.fori_loop(0, _NCHUNK // 2, pair_body,
                        jnp.zeros((16,), jnp.int32))
    wait_out(0)
    wait_out(1)
    cntbuf[...] = acc
    pltpu.sync_copy(cntbuf, cnt_hbm.at[wid])


_sc_call = functools.partial(
    pl.kernel,
    out_type=[
        jax.ShapeDtypeStruct((_RS,), jnp.int32),
        jax.ShapeDtypeStruct((_RS * 4,), jnp.float32),
        jax.ShapeDtypeStruct((_NW, 16), jnp.int32),
    ],
    mesh=plsc.VectorSubcoreMesh(core_axis_name="c", subcore_axis_name="s"),
    compiler_params=pltpu.CompilerParams(needs_layout_passes=False),
    scratch_types=[
        pltpu.VMEM((_CH * _C,), jnp.float32),
        pltpu.VMEM((_CH * _C,), jnp.float32),
        pltpu.VMEM((_CH * 4,), jnp.float32),
        pltpu.VMEM((_CH * 4,), jnp.float32),
        pltpu.VMEM((_CH,), jnp.int32),
        pltpu.VMEM((_CH,), jnp.int32),
        pltpu.VMEM((_CH * 4,), jnp.float32),
        pltpu.VMEM((_CH * 4,), jnp.float32),
        pltpu.VMEM((16,), jnp.int32),
        pltpu.SemaphoreType.DMA,
        pltpu.SemaphoreType.DMA,
        pltpu.SemaphoreType.DMA,
        pltpu.SemaphoreType.DMA,
    ],
)(_sc_body)

_RT = _R - _RS
_BR = 2048
_NB = _RT // _BR
_GR = _BR // 128


def _tc_body(lg_ref, bx_ref, lab_ref, bout_ref, cnt_ref):
    x0 = lg_ref[:, 0:128]
    x1 = lg_ref[:, 128:256]
    which = x1 > x0
    h = jnp.maximum(x0, x1)
    ii = lax.broadcasted_iota(jnp.int32, (_BR, 128), 1)
    pos = jnp.where(which, ii + 128, ii)
    m = jnp.max(h, axis=1, keepdims=True)
    cand = jnp.where(h >= m, pos, _C)
    a = jnp.min(cand, axis=1)
    valid = m[:, 0] > 0.0
    lab = jnp.where(valid, a, -1)
    lab_ref[...] = lab.reshape(1, 1, _BR)
    c = jnp.sum(valid.astype(jnp.int32))
    cnt_ref[...] = jnp.broadcast_to(c, (1, 1, 128))

    v16 = jnp.where(lab.reshape(_GR, 128) >= 0, 1.0, 0.0)
    ei = lax.broadcasted_iota(jnp.int32, (128, 512), 0)
    el = lax.broadcasted_iota(jnp.int32, (128, 512), 1)
    e = (lax.shift_right_logical(el, 2) == ei).astype(jnp.float32)
    mask = jax.lax.dot_general(v16, e, (((1,), (0,)), ((), ())),
                               preferred_element_type=jnp.float32)
    bout_ref[...] = jnp.where(mask > 0.5, bx_ref[...], 0.0)


def _make_tc_call(interpret=False):
    return pl.pallas_call(
        _tc_body,
        grid=(_NB,),
        in_specs=[
            pl.BlockSpec((_BR, _C), lambda i: (i, 0)),
            pl.BlockSpec((_GR, 512), lambda i: (i, 0)),
        ],
        out_specs=[
            pl.BlockSpec((1, 1, _BR), lambda i: (i, 0, 0)),
            pl.BlockSpec((_GR, 512), lambda i: (i, 0)),
            pl.BlockSpec((1, 1, 128), lambda i: (i, 0, 0)),
        ],
        out_shape=[
            jax.ShapeDtypeStruct((_NB, 1, _BR), jnp.int32),
            jax.ShapeDtypeStruct((_RT // 128, 512), jnp.float32),
            jax.ShapeDtypeStruct((_NB, 1, 128), jnp.int32),
        ],
        compiler_params=pltpu.CompilerParams(
            dimension_semantics=("arbitrary",),
        ),
        interpret=interpret,
    )


_tc_call = _make_tc_call()


@jax.jit
def kernel(pred_logits, pred_boxes):
    lg = pred_logits.reshape(_R, _C)
    bx = pred_boxes.reshape(_R, 4)
    sc_lab, sc_box, sc_cnt = _sc_call(
        lg[:_RS].reshape(_RS * _C), bx[:_RS].reshape(_RS * 4))
    tc_lab, tc_box, tc_cnt = _tc_call(
        lg[_RS:], bx[_RS:].reshape(_RT // 128, 512))
    labels = jnp.concatenate(
        [sc_lab, tc_lab.reshape(_RT)]).reshape(_B, _Q)
    boxes = jnp.concatenate(
        [sc_box.reshape(_RS, 4), tc_box.reshape(_RT, 4)]).reshape(_B, _Q, 4)
    total = sc_cnt[:, 0].sum() + tc_cnt[:, 0, 0].sum()
    num_boxes = jnp.maximum(total.astype(jnp.float32), 1.0)
    return labels, boxes, num_boxes


# speedup vs baseline: 1.1784x; 1.1784x over previous
import functools

import jax
import jax.numpy as jnp
from jax import lax
from jax.experimental import pallas as pl
from jax.experimental.pallas import tpu as pltpu
from jax.experimental.pallas import tpu_sc as plsc

_B, _Q, _C = 64, 2048, 256
_R = _B * _Q

_NC, _NS = 2, 16
_NW = _NC * _NS
_RS = 16384
_RW = _RS // _NW
_CH = 128
_NCHUNK = _RW // _CH
_GROUPS = _CH // 16
_UNROLL = 4


def _sc_body(lg_hbm, bx_hbm, lab_hbm, bout_hbm, cnt_hbm,
             lbuf_a, lbuf_b, bxbuf_a, bxbuf_b, labbuf_a, labbuf_b,
             boutbuf_a, boutbuf_b, cntbuf,
             sem_in0, sem_in1, sem_out0, sem_out1):
    cid = lax.axis_index("c")
    sid = lax.axis_index("s")
    wid = sid * _NC + cid
    base_row = wid * _RW

    iot = lax.iota(jnp.int32, 16)
    riot = lax.shift_right_logical(iot, 2)
    neg_inf = jnp.full((16,), -jnp.inf, jnp.float32)
    zero_f = jnp.zeros((16,), jnp.float32)
    zero_i = jnp.zeros((16,), jnp.int32)
    neg1 = jnp.full((16,), -1, jnp.int32)

    sem_in = (sem_in0, sem_in1)
    sem_out = (sem_out0, sem_out1)
    lbufs = (lbuf_a, lbuf_b)
    bxbufs = (bxbuf_a, bxbuf_b)
    labbufs = (labbuf_a, labbuf_b)
    boutbufs = (boutbuf_a, boutbuf_b)

    def start_in(chunk, b):
        row0 = base_row + chunk * _CH
        pltpu.async_copy(lg_hbm.at[pl.ds(row0 * _C, _CH * _C)],
                         lbufs[b], sem_in[b])
        pltpu.async_copy(bx_hbm.at[pl.ds(row0 * 4, _CH * 4)],
                         bxbufs[b], sem_in[b])

    def wait_in(b):
        pltpu.make_async_copy(lg_hbm.at[pl.ds(0, _CH * _C)],
                              lbufs[b], sem_in[b]).wait()
        pltpu.make_async_copy(bx_hbm.at[pl.ds(0, _CH * 4)],
                              bxbufs[b], sem_in[b]).wait()

    def start_out(chunk, b):
        row0 = base_row + chunk * _CH
        pltpu.async_copy(labbufs[b], lab_hbm.at[pl.ds(row0, _CH)],
                         sem_out[b])
        pltpu.async_copy(boutbufs[b], bout_hbm.at[pl.ds(row0 * 4, _CH * 4)],
                         sem_out[b])

    def wait_out(b):
        pltpu.make_async_copy(labbufs[b], lab_hbm.at[pl.ds(0, _CH)],
                              sem_out[b]).wait()
        pltpu.make_async_copy(boutbufs[b], bout_hbm.at[pl.ds(0, _CH * 4)],
                              sem_out[b]).wait()

    start_in(0, 0)
    start_in(1, 1)

    def pair_body(ci2, acc):
        for b in range(2):
            chunk = ci2 * 2 + b
            lbuf = lbufs[b]
            labbuf = labbufs[b]
            boutbuf = boutbufs[b]
            bxbuf = bxbufs[b]

            wait_in(b)
            @pl.when(ci2 > 0)
            def _():
                wait_out(b)

            bvecs = [(g * 16 + iot) * _C for g in range(_GROUPS)]
            init = tuple((neg_inf, bvecs[g], bvecs[g])
                         for g in range(_GROUPS))

            def j_body(_, carry):
                out = []
                for g in range(_GROUPS):
                    best, bidxf, idxv = carry[g]
                    for _u in range(_UNROLL):
                        v = plsc.load_gather(lbuf, [idxv])
                        upd = v > best
                        best = jnp.where(upd, v, best)
                        bidxf = jnp.where(upd, idxv, bidxf)
                        idxv = idxv + 1
                    out.append((best, bidxf, idxv))
                return tuple(out)

            carry = lax.fori_loop(0, _C // _UNROLL, j_body, init)

            for g in range(_GROUPS):
                best, bidxf, _ = carry[g]
                cls = bidxf - bvecs[g]
                valid = best > zero_f
                labbuf[pl.ds(g * 16, 16)] = jnp.where(valid, cls, neg1)
                acc = acc + plsc.all_reduce_population_count(valid)

                for i in range(4):
                    ridx = (g * 16 + 4 * i) + riot
                    lv = plsc.load_gather(labbuf, [ridx])
                    bx = bxbuf[pl.ds(g * 64 + i * 16, 16)]
                    boutbuf[pl.ds(g * 64 + i * 16, 16)] = jnp.where(
                        lv >= zero_i, bx, zero_f)

            start_out(chunk, b)

            @pl.when(chunk + 2 < _NCHUNK)
            def _():
                start_in(chunk + 2, b)
        return acc

    acc = lax.fori_loop(0, _NCHUNK // 2, pair_body,
                        jnp.zeros((16,), jnp.int32))
    wait_out(0)
    wait_out(1)
    cntbuf[...] = acc
    pltpu.sync_copy(cntbuf, cnt_hbm.at[wid])


_sc_call = functools.partial(
    pl.kernel,
    out_type=[
        jax.ShapeDtypeStruct((_RS,), jnp.int32),
        jax.ShapeDtypeStruct((_RS * 4,), jnp.float32),
        jax.ShapeDtypeStruct((_NW, 16), jnp.int32),
    ],
    mesh=plsc.VectorSubcoreMesh(core_axis_name="c", subcore_axis_name="s"),
    compiler_params=pltpu.CompilerParams(needs_layout_passes=False),
    scratch_types=[
        pltpu.VMEM((_CH * _C,), jnp.float32),
        pltpu.VMEM((_CH * _C,), jnp.float32),
        pltpu.VMEM((_CH * 4,), jnp.float32),
        pltpu.VMEM((_CH * 4,), jnp.float32),
        pltpu.VMEM((_CH,), jnp.int32),
        pltpu.VMEM((_CH,), jnp.int32),
        pltpu.VMEM((_CH * 4,), jnp.float32),
        pltpu.VMEM((_CH * 4,), jnp.float32),
        pltpu.VMEM((16,), jnp.int32),
        pltpu.SemaphoreType.DMA,
        pltpu.SemaphoreType.DMA,
        pltpu.SemaphoreType.DMA,
        pltpu.SemaphoreType.DMA,
    ],
)(_sc_body)

_RT = _R - _RS
_BR = 2048
_NB = _RT // _BR
_GR = _BR // 128


def _tc_body(lg_ref, bx_ref, lab_ref, bout_ref, cnt_ref):
    x0 = lg_ref[:, 0:128]
    x1 = lg_ref[:, 128:256]
    which = x1 > x0
    h = jnp.maximum(x0, x1)
    ii = lax.broadcasted_iota(jnp.int32, (_BR, 128), 1)
    pos = jnp.where(which, ii + 128, ii)
    m = jnp.max(h, axis=1, keepdims=True)
    cand = jnp.where(h >= m, pos, _C)
    a = jnp.min(cand, axis=1)
    valid = m[:, 0] > 0.0
    lab = jnp.where(valid, a, -1)
    lab_ref[...] = lab.reshape(1, 1, _BR)
    c = jnp.sum(valid.astype(jnp.int32))
    cnt_ref[...] = jnp.broadcast_to(c, (1, 1, 128))

    bout_ref[...] = jnp.where(valid[:, None], bx_ref[...], 0.0)


def _make_tc_call(interpret=False):
    return pl.pallas_call(
        _tc_body,
        grid=(_NB,),
        in_specs=[
            pl.BlockSpec((_BR, _C), lambda i: (i, 0)),
            pl.BlockSpec((_BR, 4), lambda i: (i, 0)),
        ],
        out_specs=[
            pl.BlockSpec((1, 1, _BR), lambda i: (i, 0, 0)),
            pl.BlockSpec((_BR, 4), lambda i: (i, 0)),
            pl.BlockSpec((1, 1, 128), lambda i: (i, 0, 0)),
        ],
        out_shape=[
            jax.ShapeDtypeStruct((_NB, 1, _BR), jnp.int32),
            jax.ShapeDtypeStruct((_RT, 4), jnp.float32),
            jax.ShapeDtypeStruct((_NB, 1, 128), jnp.int32),
        ],
        compiler_params=pltpu.CompilerParams(
            dimension_semantics=("arbitrary",),
        ),
        interpret=interpret,
    )


_tc_call = _make_tc_call()


@jax.jit
def kernel(pred_logits, pred_boxes):
    lg = pred_logits.reshape(_R, _C)
    bx = pred_boxes.reshape(_R, 4)
    sc_lab, sc_box, sc_cnt = _sc_call(
        lg[:_RS].reshape(_RS * _C), bx[:_RS].reshape(_RS * 4))
    tc_lab, tc_box, tc_cnt = _tc_call(lg[_RS:], bx[_RS:])
    labels = jnp.concatenate(
        [sc_lab, tc_lab.reshape(_RT)]).reshape(_B, _Q)
    boxes = jnp.concatenate(
        [sc_box.reshape(_RS, 4), tc_box]).reshape(_B, _Q, 4)
    total = sc_cnt[:, 0].sum() + tc_cnt[:, 0, 0].sum()
    num_boxes = jnp.maximum(total.astype(jnp.float32), 1.0)
    return labels, boxes, num_boxes
